# baseline (speedup 1.0000x reference)
import functools

import jax
import jax.numpy as jnp
from jax import lax
from jax.experimental import pallas as pl
from jax.experimental.pallas import tpu as pltpu
from jax.experimental.pallas import tpu_sc as plsc

VOCAB = 1000000
EMBED_DIM = 64
BATCH = 4096
HIST = 200

NC = 2
NS = 16
NW = NC * NS

CHUNK = 128
NCHUNK = 7813
FULL_K = (NCHUNK - 1) // NW
SROWS = 500032

GROUP = 128
BCHUNK = BATCH // GROUP
NGROUP = HIST * BCHUNK
PER_W = NGROUP // NW

_f32 = jnp.float32
_i32 = jnp.int32


def _transpose_128(src, dst, n_j):
    evecs = [lax.iota(_i32, 16) + e0 * 16 for e0 in range(4)]
    for j in range(n_j):
        jsplat = jnp.full((16,), j, _i32)
        for e0 in range(4):
            v = plsc.load_gather(src, [evecs[e0], jsplat])
            dst[j // 2, pl.ds((j % 2) * 64 + e0 * 16, 16)] = v


def _make_transpose():
    mesh = plsc.VectorSubcoreMesh(core_axis_name="c", subcore_axis_name="s")

    @functools.partial(
        pl.kernel,
        out_type=jax.ShapeDtypeStruct((SROWS, 128), _f32),
        mesh=mesh,
        scratch_types=[
            pltpu.VMEM((2, 64, 128), _f32),
            pltpu.VMEM((2, 64, 128), _f32),
            pltpu.VMEM((64, 64), _f32),
            pltpu.SemaphoreType.DMA((2,)),
            pltpu.SemaphoreType.DMA((2,)),
        ],
        compiler_params=pltpu.CompilerParams(use_tc_tiling_on_sc=True, needs_layout_passes=False),
    )
    def transpose_kernel(tt_hbm, tail_hbm, scr_hbm, vin, vout, tail_v,
                         isem, osem):
        wid = lax.axis_index("s") * NC + lax.axis_index("c")

        def chunk_of(k):
            return k * NW + wid

        def start_in(k, b):
            ct = chunk_of(k)
            pltpu.async_copy(tt_hbm.at[:, pl.ds(ct * CHUNK, CHUNK)],
                             vin.at[b], isem.at[b])

        def wait_in(b):
            pltpu.make_async_copy(tt_hbm.at[:, pl.ds(0, CHUNK)],
                                  vin.at[b], isem.at[b]).wait()

        def start_out(k, b):
            ct = chunk_of(k)
            pltpu.async_copy(vout.at[b],
                             scr_hbm.at[pl.ds(ct * 64, 64)], osem.at[b])

        def wait_out(b):
            pltpu.make_async_copy(vout.at[b],
                                  scr_hbm.at[pl.ds(0, 64)], osem.at[b]).wait()

        last = chunk_of(FULL_K)
        has_extra = last < NCHUNK - 1

        start_in(0, 0)
        start_in(1, 1)

        def body(i, _):
            k0 = i * 2
            for p in range(2):
                k = k0 + p

                @pl.when(k >= 2)
                def _():
                    wait_out(p)

                wait_in(p)
                _transpose_128(vin.at[p], vout.at[p], 128)
                start_out(k, p)

                @pl.when((k + 2 < FULL_K) | ((k + 2 == FULL_K) & has_extra))
                def _():
                    start_in(k + 2, p)

            return ()

        lax.fori_loop(0, FULL_K // 2, body, (), unroll=False)

        @pl.when(last < NCHUNK - 1)
        def _():
            wait_in(0)
            wait_out(0)
            _transpose_128(vin.at[0], vout.at[0], 128)
            start_out(FULL_K, 0)
            wait_out(0)

        @pl.when(last == NCHUNK - 1)
        def _():
            pltpu.sync_copy(tail_hbm, tail_v)
            wait_out(0)
            _transpose_128(tail_v, vout.at[0], 64)
            start_out(FULL_K, 0)
            wait_out(0)

        @pl.when(last > NCHUNK - 1)
        def _():
            wait_out(0)

        wait_out(1)

    return transpose_kernel


def _make_gather():
    mesh = plsc.VectorSubcoreMesh(core_axis_name="c", subcore_axis_name="s")

    @functools.partial(
        pl.kernel,
        out_type=jax.ShapeDtypeStruct((HIST, EMBED_DIM, BATCH), _f32),
        mesh=mesh,
        scratch_types=[
            pltpu.VMEM((PER_W, GROUP), _i32),
            pltpu.VMEM((PER_W, GROUP), _i32),
            pltpu.VMEM((2, GROUP, 128), _f32),
            pltpu.VMEM((2, EMBED_DIM, GROUP), _f32),
            pltpu.SemaphoreType.DMA((2,)),
            pltpu.SemaphoreType.DMA((2,)),
        ],
        compiler_params=pltpu.CompilerParams(use_tc_tiling_on_sc=True, needs_layout_passes=False),
    )
    def gather_kernel(idx_hbm, scr_hbm, out_hbm, idx_v, half_v, rows_v,
                      trans_v, gsem, osem):
        wid = lax.axis_index("s") * NC + lax.axis_index("c")
        pltpu.sync_copy(idx_hbm.at[wid], idx_v)

        def shift_body(g, _):
            for q in range(8):
                v = idx_v[g, pl.ds(q * 16, 16)]
                half_v[g, pl.ds(q * 16, 16)] = lax.shift_right_logical(v, 1)
            return ()

        lax.fori_loop(0, PER_W, shift_body, (), unroll=False)

        def start_gather(g, b):
            pltpu.async_copy(scr_hbm.at[half_v.at[g]], rows_v.at[b],
                             gsem.at[b])

        def wait_gather(b):
            pltpu.make_async_copy(scr_hbm.at[pl.ds(0, GROUP)],
                                  rows_v.at[b], gsem.at[b]).wait()

        def start_out(g, b):
            gid = wid * PER_W + g
            h = gid // BCHUNK
            b0 = lax.rem(gid, BCHUNK) * GROUP
            pltpu.async_copy(trans_v.at[b],
                             out_hbm.at[h, :, pl.ds(b0, GROUP)], osem.at[b])

        def wait_out(b):
            pltpu.make_async_copy(trans_v.at[b],
                                  out_hbm.at[0, :, pl.ds(0, GROUP)],
                                  osem.at[b]).wait()

        jvecs = [lax.iota(_i32, 16) + q * 16 for q in range(8)]

        start_gather(0, 0)
        start_gather(1, 1)

        def body(i, _):
            g0 = i * 2
            for p in range(2):
                g = g0 + p
                wait_gather(p)

                @pl.when(g >= 2)
                def _():
                    wait_out(p)

                offs = []
                for q in range(8):
                    par = lax.bitwise_and(idx_v[g, pl.ds(q * 16, 16)],
                                          jnp.full((16,), 1, _i32))
                    offs.append(par * 64)
                src = rows_v.at[p]
                dst = trans_v.at[p]
                for e in range(EMBED_DIM):
                    esplat = jnp.full((16,), e, _i32)
                    for q in range(8):
                        v = plsc.load_gather(src, [jvecs[q], offs[q] + esplat])
                        dst[e, pl.ds(q * 16, 16)] = v
                start_out(g, p)

                @pl.when(g + 2 < PER_W)
                def _():
                    start_gather(g + 2, p)

            return ()

        lax.fori_loop(0, PER_W // 2, body, (), unroll=False)
        wait_out(0)
        wait_out(1)

    return gather_kernel


_transpose = _make_transpose()
_gather = _make_gather()


def kernel(batch, table):
    tt = table.T
    tail = table.T[:, VOCAB - 64:]
    scratch = _transpose(tt, tail)
    idx = batch.astype(_i32).T.reshape(NW, PER_W, GROUP)
    o3 = _gather(idx, scratch)
    return o3.transpose(2, 0, 1)


# speedup vs baseline: 3.3497x; 3.3497x over previous
import functools

import jax
import jax.numpy as jnp
from jax import lax
from jax.experimental import pallas as pl
from jax.experimental.pallas import tpu as pltpu
from jax.experimental.pallas import tpu_sc as plsc

VOCAB = 1000000
EMBED_DIM = 64
BATCH = 4096
HIST = 200

NC = 2
NS = 16
NW = NC * NS

TOTAL = BATCH * HIST
PER_W = TOTAL // NW
GROUP = 128
GROUPS = PER_W // GROUP
NBUF = 8
LEAD = 4


def _make_gather():
    mesh = plsc.VectorSubcoreMesh(core_axis_name="c", subcore_axis_name="s")

    @functools.partial(
        pl.kernel,
        out_type=jax.ShapeDtypeStruct((TOTAL, 128), jnp.float32),
        mesh=mesh,
        scratch_types=[
            pltpu.VMEM((GROUPS, GROUP), jnp.int32),
            pltpu.VMEM((NBUF, GROUP, EMBED_DIM), jnp.float32),
            pltpu.SemaphoreType.DMA((NBUF,)),
            pltpu.SemaphoreType.DMA((NBUF,)),
        ],
        compiler_params=pltpu.CompilerParams(use_tc_tiling_on_sc=False),
    )
    def gather_kernel(idx_hbm, table_hbm, out_hbm, idx_v, rows_v, gsem, ssem):
        wid = lax.axis_index("s") * NC + lax.axis_index("c")
        base = wid * PER_W
        pltpu.sync_copy(idx_hbm.at[wid], idx_v)

        def start_gather(g, b):
            pltpu.async_copy(table_hbm.at[idx_v.at[g]], rows_v.at[b],
                             gsem.at[b])

        def wait_gather(b):
            pltpu.make_async_copy(table_hbm.at[pl.ds(0, GROUP)],
                                  rows_v.at[b], gsem.at[b]).wait()

        def start_store(g, b):
            pltpu.async_copy(rows_v.at[b],
                             out_hbm.at[pl.ds(base + g * GROUP, GROUP),
                                        pl.ds(0, EMBED_DIM)],
                             ssem.at[b])

        def wait_store(b):
            pltpu.make_async_copy(rows_v.at[b],
                                  out_hbm.at[pl.ds(base, GROUP),
                                             pl.ds(0, EMBED_DIM)],
                                  ssem.at[b]).wait()

        for g in range(LEAD):
            start_gather(g, g % NBUF)

        def outer(i, _):
            t0 = i * NBUF
            for j in range(NBUF):
                t = t0 + j
                u = t + LEAD
                bu = (j + LEAD) % NBUF

                @pl.when(u < GROUPS)
                def _():
                    @pl.when(u >= NBUF)
                    def _():
                        wait_store(bu)
                    start_gather(u, bu)

                wait_gather(j)
                start_store(t, j)
            return ()

        lax.fori_loop(0, GROUPS // NBUF, outer, (), unroll=False)

        for b in range(NBUF):
            wait_store(b)

    return gather_kernel


_gather = _make_gather()


def kernel(batch, table):
    idx = batch.astype(jnp.int32).reshape(NW, GROUPS, GROUP)
    flat = _gather(idx, table)
    return flat[:, :EMBED_DIM].reshape(BATCH, HIST, EMBED_DIM)
